# Initial kernel scaffold; baseline (speedup 1.0000x reference)
import jax
import jax.numpy as jnp
from jax.experimental import pallas as pl


def kernel(x, edge_index, W, b, gamma, beta, fcW, fcb):
    raise NotImplementedError("write your pallas kernel here")


# speedup vs baseline: 22.3771x; 22.3771x over previous
import functools

import jax
import jax.numpy as jnp
from jax import lax
from jax.experimental import pallas as pl
from jax.experimental.pallas import tpu as pltpu
from jax.experimental.pallas import tpu_sc as plsc

_LANES = 128
_NW = 32


def _pad_up(n, m):
    return (n + m - 1) // m * m


def _sc_deg_kernel(np_, epw, chunks, stretch):
    mesh = plsc.VectorSubcoreMesh(core_axis_name="c", subcore_axis_name="s")

    @functools.partial(
        pl.kernel,
        out_type=jax.ShapeDtypeStruct((2 * np_,), jnp.float32),
        mesh=mesh,
        scratch_types=[
            pltpu.VMEM((_LANES,), jnp.int32),
            pltpu.VMEM((_LANES,), jnp.float32),
            pltpu.VMEM_SHARED((np_,), jnp.float32),
        ],
        compiler_params=pltpu.CompilerParams(use_tc_tiling_on_sc=False),
    )
    def deg_kernel(dst_hbm, zeros_hbm, out_hbm, idx_v, ones_v, deg_sh):
        c = lax.axis_index("c")
        s = lax.axis_index("s")
        pltpu.sync_copy(zeros_hbm.at[pl.ds(s * stretch, stretch)],
                        deg_sh.at[pl.ds(s * stretch, stretch)])
        for j in range(_LANES // 16):
            ones_v[pl.ds(j * 16, 16)] = jnp.ones((16,), jnp.float32)
        plsc.subcore_barrier()
        w = c * 16 + s
        ebase = w * epw

        def body(i, carry):
            pltpu.sync_copy(dst_hbm.at[pl.ds(ebase + i * _LANES, _LANES)], idx_v)
            pltpu.sync_copy(ones_v, deg_sh.at[idx_v], add=True)
            return carry

        lax.fori_loop(0, chunks, body, 0)
        plsc.subcore_barrier()
        pltpu.sync_copy(deg_sh.at[pl.ds(s * stretch, stretch)],
                        out_hbm.at[pl.ds(c * np_ + s * stretch, stretch)])

    return deg_kernel


def _sc_msg_kernel(np_, epw, chunks, stretch):
    mesh = plsc.VectorSubcoreMesh(core_axis_name="c", subcore_axis_name="s")

    @functools.partial(
        pl.kernel,
        out_type=jax.ShapeDtypeStruct((4 * np_,), jnp.float32),
        mesh=mesh,
        scratch_types=[
            pltpu.VMEM((_LANES,), jnp.int32),
            pltpu.VMEM((_LANES,), jnp.int32),
            pltpu.VMEM((_LANES,), jnp.float32),
            pltpu.VMEM((_LANES,), jnp.float32),
            pltpu.VMEM_SHARED((np_,), jnp.float32),
            pltpu.VMEM_SHARED((np_,), jnp.float32),
            pltpu.SemaphoreType.DMA,
            pltpu.SemaphoreType.DMA,
        ],
        compiler_params=pltpu.CompilerParams(use_tc_tiling_on_sc=False),
    )
    def msg_kernel(src_hbm, dst_hbm, y0_hbm, y1_hbm, zeros_hbm, out_hbm,
                   sidx, didx, g0, g1, s0_sh, s1_sh, sem0, sem1):
        c = lax.axis_index("c")
        s = lax.axis_index("s")
        pltpu.sync_copy(zeros_hbm.at[pl.ds(s * stretch, stretch)],
                        s0_sh.at[pl.ds(s * stretch, stretch)])
        pltpu.sync_copy(zeros_hbm.at[pl.ds(s * stretch, stretch)],
                        s1_sh.at[pl.ds(s * stretch, stretch)])
        plsc.subcore_barrier()
        w = c * 16 + s
        ebase = w * epw

        def body(i, carry):
            pltpu.sync_copy(src_hbm.at[pl.ds(ebase + i * _LANES, _LANES)], sidx)
            pltpu.sync_copy(dst_hbm.at[pl.ds(ebase + i * _LANES, _LANES)], didx)
            d0 = pltpu.async_copy(y0_hbm.at[sidx], g0, sem0)
            d1 = pltpu.async_copy(y1_hbm.at[sidx], g1, sem1)
            d0.wait()
            d1.wait()
            pltpu.sync_copy(g0, s0_sh.at[didx], add=True)
            pltpu.sync_copy(g1, s1_sh.at[didx], add=True)
            return carry

        lax.fori_loop(0, chunks, body, 0)
        plsc.subcore_barrier()
        base = c * 2 * np_
        pltpu.sync_copy(s0_sh.at[pl.ds(s * stretch, stretch)],
                        out_hbm.at[pl.ds(base + s * stretch, stretch)])
        pltpu.sync_copy(s1_sh.at[pl.ds(s * stretch, stretch)],
                        out_hbm.at[pl.ds(base + np_ + s * stretch, stretch)])

    return msg_kernel


def kernel(x, edge_index, W, b, gamma, beta, fcW, fcb):
    n, in_dim = x.shape
    out_dim = W.shape[1]
    e = edge_index.shape[1]

    np_ = _pad_up(n + 1, 2048)
    stretch = np_ // 16
    e_pad = _pad_up(e, _NW * _LANES)
    epw = e_pad // _NW
    chunks = epw // _LANES

    src = edge_index[0].astype(jnp.int32)
    dst = edge_index[1].astype(jnp.int32)
    pad = e_pad - e
    if pad:
        src = jnp.concatenate([src, jnp.zeros((pad,), jnp.int32)])
        dst = jnp.concatenate([dst, jnp.full((pad,), n, jnp.int32)])

    zeros1 = jnp.zeros((np_,), jnp.float32)

    deg_flat = _sc_deg_kernel(np_, epw, chunks, stretch)(dst, zeros1)
    deg_parts = deg_flat.reshape(2, np_)

    xT = jnp.pad(x.T, ((0, 0), (0, np_ - n)))

    def prep_body(degp_ref, xT_ref, yT_ref, dinv_ref):
        dsum = degp_ref[0:1, :] + degp_ref[1:2, :] + 1.0
        dinv = lax.rsqrt(dsum)
        dinv_ref[...] = dinv
        yT_ref[...] = xT_ref[...] * dinv

    yT, dinv_row = pl.pallas_call(
        prep_body,
        out_shape=[jax.ShapeDtypeStruct((in_dim, np_), jnp.float32),
                   jax.ShapeDtypeStruct((1, np_), jnp.float32)],
    )(deg_parts, xT)
    y0 = yT[0]
    y1 = yT[1]

    s_flat = _sc_msg_kernel(np_, epw, chunks, stretch)(src, dst, y0, y1, zeros1)
    s00 = s_flat[0 * np_:1 * np_].reshape(np_, 1)
    s01 = s_flat[1 * np_:2 * np_].reshape(np_, 1)
    s10 = s_flat[2 * np_:3 * np_].reshape(np_, 1)
    s11 = s_flat[3 * np_:4 * np_].reshape(np_, 1)
    y0c = y0.reshape(np_, 1)
    y1c = y1.reshape(np_, 1)
    dinv_col = dinv_row.T

    blk = 2000
    nblk = n // blk
    b_row = b.reshape(1, out_dim)
    g_row = gamma.reshape(1, out_dim)
    be_row = beta.reshape(1, out_dim)
    fcb_row = fcb.reshape(1, out_dim)

    def _h_block(s00r, s01r, s10r, s11r, y0r, y1r, dr, Wr, br):
        a0 = (s00r[...] + s10r[...] + y0r[...]) * dr[...]
        a1 = (s01r[...] + s11r[...] + y1r[...]) * dr[...]
        h = a0 * Wr[0:1, :] + a1 * Wr[1:2, :] + br[...]
        return jnp.maximum(h, 0.0)

    def stats_body(s00r, s01r, s10r, s11r, y0r, y1r, dr, Wr, br,
                   mean_o, var_o, acc_s, acc_q):
        i = pl.program_id(0)
        h = _h_block(s00r, s01r, s10r, s11r, y0r, y1r, dr, Wr, br)
        ps = jnp.sum(h, axis=0, keepdims=True)
        pq = jnp.sum(h * h, axis=0, keepdims=True)

        @pl.when(i == 0)
        def _():
            acc_s[...] = ps
            acc_q[...] = pq

        @pl.when(i > 0)
        def _():
            acc_s[...] += ps
            acc_q[...] += pq

        @pl.when(i == nblk - 1)
        def _():
            m = acc_s[...] * (1.0 / n)
            mean_o[...] = m
            var_o[...] = acc_q[...] * (1.0 / n) - m * m

    col_spec = pl.BlockSpec((blk, 1), lambda i: (i, 0))
    full = lambda shape: pl.BlockSpec(shape, lambda i: (0, 0))
    col_ins = [col_spec] * 7
    mean, var = pl.pallas_call(
        stats_body,
        grid=(nblk,),
        in_specs=col_ins + [full((in_dim, out_dim)), full((1, out_dim))],
        out_specs=[full((1, out_dim)), full((1, out_dim))],
        out_shape=[jax.ShapeDtypeStruct((1, out_dim), jnp.float32),
                   jax.ShapeDtypeStruct((1, out_dim), jnp.float32)],
        scratch_shapes=[pltpu.VMEM((1, out_dim), jnp.float32),
                        pltpu.VMEM((1, out_dim), jnp.float32)],
        compiler_params=pltpu.CompilerParams(
            dimension_semantics=("arbitrary",)),
    )(s00, s01, s10, s11, y0c, y1c, dinv_col, W, b_row)

    def out_body(s00r, s01r, s10r, s11r, y0r, y1r, dr, Wr, br, mr, vr,
                 gr, ber, fcWr, fcbr, o1, o2):
        h = _h_block(s00r, s01r, s10r, s11r, y0r, y1r, dr, Wr, br)
        scale = gr[...] * lax.rsqrt(vr[...] + 1e-5)
        hn = (h - mr[...]) * scale + ber[...]
        hf = jnp.dot(hn, fcWr[...], preferred_element_type=jnp.float32)
        hf = hf + fcbr[...]
        o2[...] = hf
        m = jnp.max(hf, axis=1, keepdims=True)
        shifted = hf - m
        lse = jnp.log(jnp.sum(jnp.exp(shifted), axis=1, keepdims=True))
        o1[...] = shifted - lse

    out_spec = pl.BlockSpec((blk, out_dim), lambda i: (i, 0))
    out1, out2 = pl.pallas_call(
        out_body,
        grid=(nblk,),
        in_specs=col_ins + [full((in_dim, out_dim)), full((1, out_dim)),
                            full((1, out_dim)), full((1, out_dim)),
                            full((1, out_dim)), full((1, out_dim)),
                            full((out_dim, out_dim)), full((1, out_dim))],
        out_specs=[out_spec, out_spec],
        out_shape=[jax.ShapeDtypeStruct((n, out_dim), jnp.float32),
                   jax.ShapeDtypeStruct((n, out_dim), jnp.float32)],
        compiler_params=pltpu.CompilerParams(
            dimension_semantics=("arbitrary",)),
    )(s00, s01, s10, s11, y0c, y1c, dinv_col, W, b_row, mean, var,
      g_row, be_row, fcW, fcb_row)
    return out1, out2


# speedup vs baseline: 40.2414x; 1.7983x over previous
import functools

import jax
import jax.numpy as jnp
from jax import lax
from jax.experimental import pallas as pl
from jax.experimental.pallas import tpu as pltpu
from jax.experimental.pallas import tpu_sc as plsc

_LANES = 128
_NW = 32


def _pad_up(n, m):
    return (n + m - 1) // m * m


_K = 4


def _sc_deg_kernel(np_, epw, chunks, stretch):
    mesh = plsc.VectorSubcoreMesh(core_axis_name="c", subcore_axis_name="s")
    ngroups = chunks // _K

    @functools.partial(
        pl.kernel,
        out_type=jax.ShapeDtypeStruct((2 * np_,), jnp.float32),
        mesh=mesh,
        scratch_types=[
            pltpu.VMEM((_K, _LANES), jnp.int32),
            pltpu.VMEM((_K, _LANES), jnp.int32),
            pltpu.VMEM((_LANES,), jnp.float32),
            pltpu.VMEM_SHARED((np_,), jnp.float32),
            pltpu.SemaphoreType.DMA,
            pltpu.SemaphoreType.DMA,
        ],
        compiler_params=pltpu.CompilerParams(use_tc_tiling_on_sc=False),
    )
    def deg_kernel(dst_hbm, zeros_hbm, out_hbm, idx_a, idx_b, ones_v,
                   deg_sh, sem_a, sem_b):
        c = lax.axis_index("c")
        s = lax.axis_index("s")
        pltpu.sync_copy(zeros_hbm.at[pl.ds(s * stretch, stretch)],
                        deg_sh.at[pl.ds(s * stretch, stretch)])
        for j in range(_LANES // 16):
            ones_v[pl.ds(j * 16, 16)] = jnp.ones((16,), jnp.float32)
        plsc.subcore_barrier()
        w = c * 16 + s
        wc0 = w * chunks

        def phase(g, cur_idx, nxt_idx, sem):
            ds = []
            for b in range(_K):
                ds.append(pltpu.async_copy(
                    ones_v, deg_sh.at[cur_idx.at[b]], sem, add=True))
            @pl.when(g + 1 < ngroups)
            def _():
                pltpu.sync_copy(dst_hbm.at[pl.ds(wc0 + (g + 1) * _K, _K)],
                                nxt_idx)
            for d in ds:
                d.wait()

        pltpu.sync_copy(dst_hbm.at[pl.ds(wc0, _K)], idx_a)

        def body(i, carry):
            phase(2 * i, idx_a, idx_b, sem_a)
            phase(2 * i + 1, idx_b, idx_a, sem_b)
            return carry

        lax.fori_loop(0, ngroups // 2, body, 0)
        plsc.subcore_barrier()
        pltpu.sync_copy(deg_sh.at[pl.ds(s * stretch, stretch)],
                        out_hbm.at[pl.ds(c * np_ + s * stretch, stretch)])

    return deg_kernel


def _sc_msg_kernel(np_, epw, chunks, stretch):
    mesh = plsc.VectorSubcoreMesh(core_axis_name="c", subcore_axis_name="s")
    ngroups = chunks // _K

    @functools.partial(
        pl.kernel,
        out_type=jax.ShapeDtypeStruct((4 * np_,), jnp.float32),
        mesh=mesh,
        scratch_types=[
            pltpu.VMEM((_K, _LANES), jnp.int32),
            pltpu.VMEM((_K, _LANES), jnp.int32),
            pltpu.VMEM((_K, _LANES), jnp.int32),
            pltpu.VMEM((_K, _LANES), jnp.int32),
            pltpu.VMEM((_K, _LANES), jnp.float32),
            pltpu.VMEM((_K, _LANES), jnp.float32),
            pltpu.VMEM((_K, _LANES), jnp.float32),
            pltpu.VMEM((_K, _LANES), jnp.float32),
            pltpu.VMEM_SHARED((np_,), jnp.float32),
            pltpu.VMEM_SHARED((np_,), jnp.float32),
            pltpu.SemaphoreType.DMA,
            pltpu.SemaphoreType.DMA,
            pltpu.SemaphoreType.DMA,
            pltpu.SemaphoreType.DMA,
        ],
        compiler_params=pltpu.CompilerParams(use_tc_tiling_on_sc=False),
    )
    def msg_kernel(src_hbm, dst_hbm, y0_hbm, y1_hbm, zeros_hbm, out_hbm,
                   sidx_a, didx_a, sidx_b, didx_b, g0a, g1a, g0b, g1b,
                   s0_sh, s1_sh, semg_a, semg_b, sems_a, sems_b):
        c = lax.axis_index("c")
        s = lax.axis_index("s")
        pltpu.sync_copy(zeros_hbm.at[pl.ds(s * stretch, stretch)],
                        s0_sh.at[pl.ds(s * stretch, stretch)])
        pltpu.sync_copy(zeros_hbm.at[pl.ds(s * stretch, stretch)],
                        s1_sh.at[pl.ds(s * stretch, stretch)])
        plsc.subcore_barrier()
        w = c * 16 + s
        wc0 = w * chunks

        def fire_gathers(sidx, g0, g1, semg):
            for b in range(_K):
                pltpu.async_copy(y0_hbm.at[sidx.at[b]], g0.at[b], semg)
                pltpu.async_copy(y1_hbm.at[sidx.at[b]], g1.at[b], semg)

        def phase(g, sidx, didx, g0, g1, semg, nsidx, ndidx, ng0, ng1, nsemg,
                  sems):
            @pl.when(g + 1 < ngroups)
            def _():
                pltpu.sync_copy(src_hbm.at[pl.ds(wc0 + (g + 1) * _K, _K)],
                                nsidx)
                pltpu.sync_copy(dst_hbm.at[pl.ds(wc0 + (g + 1) * _K, _K)],
                                ndidx)
                fire_gathers(nsidx, ng0, ng1, nsemg)
            for b in range(_K):
                pltpu.make_async_copy(y0_hbm.at[sidx.at[b]], g0.at[b],
                                      semg).wait()
                pltpu.make_async_copy(y1_hbm.at[sidx.at[b]], g1.at[b],
                                      semg).wait()
            ds = []
            for b in range(_K):
                ds.append(pltpu.async_copy(g0.at[b], s0_sh.at[didx.at[b]],
                                           sems, add=True))
                ds.append(pltpu.async_copy(g1.at[b], s1_sh.at[didx.at[b]],
                                           sems, add=True))
            for d in ds:
                d.wait()

        pltpu.sync_copy(src_hbm.at[pl.ds(wc0, _K)], sidx_a)
        pltpu.sync_copy(dst_hbm.at[pl.ds(wc0, _K)], didx_a)
        fire_gathers(sidx_a, g0a, g1a, semg_a)

        def body(i, carry):
            phase(2 * i, sidx_a, didx_a, g0a, g1a, semg_a,
                  sidx_b, didx_b, g0b, g1b, semg_b, sems_a)
            phase(2 * i + 1, sidx_b, didx_b, g0b, g1b, semg_b,
                  sidx_a, didx_a, g0a, g1a, semg_a, sems_b)
            return carry

        lax.fori_loop(0, ngroups // 2, body, 0)
        plsc.subcore_barrier()
        base = c * 2 * np_
        pltpu.sync_copy(s0_sh.at[pl.ds(s * stretch, stretch)],
                        out_hbm.at[pl.ds(base + s * stretch, stretch)])
        pltpu.sync_copy(s1_sh.at[pl.ds(s * stretch, stretch)],
                        out_hbm.at[pl.ds(base + np_ + s * stretch, stretch)])

    return msg_kernel


def kernel(x, edge_index, W, b, gamma, beta, fcW, fcb):
    n, in_dim = x.shape
    out_dim = W.shape[1]
    e = edge_index.shape[1]

    np_ = _pad_up(n + 1, 2048)
    stretch = np_ // 16
    e_pad = _pad_up(e, _NW * _LANES * _K * 2)
    epw = e_pad // _NW
    chunks = epw // _LANES

    src = edge_index[0].astype(jnp.int32)
    dst = edge_index[1].astype(jnp.int32)
    pad = e_pad - e
    if pad:
        src = jnp.concatenate([src, jnp.zeros((pad,), jnp.int32)])
        dst = jnp.concatenate([dst, jnp.full((pad,), n, jnp.int32)])
    src = src.reshape(e_pad // _LANES, _LANES)
    dst = dst.reshape(e_pad // _LANES, _LANES)

    zeros1 = jnp.zeros((np_,), jnp.float32)

    deg_flat = _sc_deg_kernel(np_, epw, chunks, stretch)(dst, zeros1)
    deg_parts = deg_flat.reshape(2, np_)

    xT = jnp.pad(x.T, ((0, 0), (0, np_ - n)))

    def prep_body(degp_ref, xT_ref, yT_ref, dinv_ref):
        dsum = degp_ref[0:1, :] + degp_ref[1:2, :] + 1.0
        dinv = lax.rsqrt(dsum)
        dinv_ref[...] = dinv
        yT_ref[...] = xT_ref[...] * dinv

    yT, dinv_row = pl.pallas_call(
        prep_body,
        out_shape=[jax.ShapeDtypeStruct((in_dim, np_), jnp.float32),
                   jax.ShapeDtypeStruct((1, np_), jnp.float32)],
    )(deg_parts, xT)
    y0 = yT[0]
    y1 = yT[1]

    s_flat = _sc_msg_kernel(np_, epw, chunks, stretch)(src, dst, y0, y1, zeros1)
    s00 = s_flat[0 * np_:1 * np_].reshape(np_, 1)
    s01 = s_flat[1 * np_:2 * np_].reshape(np_, 1)
    s10 = s_flat[2 * np_:3 * np_].reshape(np_, 1)
    s11 = s_flat[3 * np_:4 * np_].reshape(np_, 1)
    y0c = y0.reshape(np_, 1)
    y1c = y1.reshape(np_, 1)
    dinv_col = dinv_row.T

    blk = 2000
    nblk = n // blk
    b_row = b.reshape(1, out_dim)
    g_row = gamma.reshape(1, out_dim)
    be_row = beta.reshape(1, out_dim)
    fcb_row = fcb.reshape(1, out_dim)

    def _h_block(s00r, s01r, s10r, s11r, y0r, y1r, dr, Wr, br):
        a0 = (s00r[...] + s10r[...] + y0r[...]) * dr[...]
        a1 = (s01r[...] + s11r[...] + y1r[...]) * dr[...]
        h = a0 * Wr[0:1, :] + a1 * Wr[1:2, :] + br[...]
        return jnp.maximum(h, 0.0)

    def stats_body(s00r, s01r, s10r, s11r, y0r, y1r, dr, Wr, br,
                   mean_o, var_o, acc_s, acc_q):
        i = pl.program_id(0)
        h = _h_block(s00r, s01r, s10r, s11r, y0r, y1r, dr, Wr, br)
        ps = jnp.sum(h, axis=0, keepdims=True)
        pq = jnp.sum(h * h, axis=0, keepdims=True)

        @pl.when(i == 0)
        def _():
            acc_s[...] = ps
            acc_q[...] = pq

        @pl.when(i > 0)
        def _():
            acc_s[...] += ps
            acc_q[...] += pq

        @pl.when(i == nblk - 1)
        def _():
            m = acc_s[...] * (1.0 / n)
            mean_o[...] = m
            var_o[...] = acc_q[...] * (1.0 / n) - m * m

    col_spec = pl.BlockSpec((blk, 1), lambda i: (i, 0))
    full = lambda shape: pl.BlockSpec(shape, lambda i: (0, 0))
    col_ins = [col_spec] * 7
    mean, var = pl.pallas_call(
        stats_body,
        grid=(nblk,),
        in_specs=col_ins + [full((in_dim, out_dim)), full((1, out_dim))],
        out_specs=[full((1, out_dim)), full((1, out_dim))],
        out_shape=[jax.ShapeDtypeStruct((1, out_dim), jnp.float32),
                   jax.ShapeDtypeStruct((1, out_dim), jnp.float32)],
        scratch_shapes=[pltpu.VMEM((1, out_dim), jnp.float32),
                        pltpu.VMEM((1, out_dim), jnp.float32)],
        compiler_params=pltpu.CompilerParams(
            dimension_semantics=("arbitrary",)),
    )(s00, s01, s10, s11, y0c, y1c, dinv_col, W, b_row)

    def out_body(s00r, s01r, s10r, s11r, y0r, y1r, dr, Wr, br, mr, vr,
                 gr, ber, fcWr, fcbr, o1, o2):
        h = _h_block(s00r, s01r, s10r, s11r, y0r, y1r, dr, Wr, br)
        scale = gr[...] * lax.rsqrt(vr[...] + 1e-5)
        hn = (h - mr[...]) * scale + ber[...]
        hf = jnp.dot(hn, fcWr[...], preferred_element_type=jnp.float32)
        hf = hf + fcbr[...]
        o2[...] = hf
        m = jnp.max(hf, axis=1, keepdims=True)
        shifted = hf - m
        lse = jnp.log(jnp.sum(jnp.exp(shifted), axis=1, keepdims=True))
        o1[...] = shifted - lse

    out_spec = pl.BlockSpec((blk, out_dim), lambda i: (i, 0))
    out1, out2 = pl.pallas_call(
        out_body,
        grid=(nblk,),
        in_specs=col_ins + [full((in_dim, out_dim)), full((1, out_dim)),
                            full((1, out_dim)), full((1, out_dim)),
                            full((1, out_dim)), full((1, out_dim)),
                            full((out_dim, out_dim)), full((1, out_dim))],
        out_specs=[out_spec, out_spec],
        out_shape=[jax.ShapeDtypeStruct((n, out_dim), jnp.float32),
                   jax.ShapeDtypeStruct((n, out_dim), jnp.float32)],
        compiler_params=pltpu.CompilerParams(
            dimension_semantics=("arbitrary",)),
    )(s00, s01, s10, s11, y0c, y1c, dinv_col, W, b_row, mean, var,
      g_row, be_row, fcW, fcb_row)
    return out1, out2


# speedup vs baseline: 56.7559x; 1.4104x over previous
import functools

import jax
import jax.numpy as jnp
from jax import lax
from jax.experimental import pallas as pl
from jax.experimental.pallas import tpu as pltpu
from jax.experimental.pallas import tpu_sc as plsc

_LANES = 128
_NW = 32


def _pad_up(n, m):
    return (n + m - 1) // m * m


_K = 4


def _sc_deg_kernel(np_, epw, chunks, stretch):
    mesh = plsc.VectorSubcoreMesh(core_axis_name="c", subcore_axis_name="s")
    ngroups = chunks // _K

    @functools.partial(
        pl.kernel,
        out_type=jax.ShapeDtypeStruct((2 * np_,), jnp.float32),
        mesh=mesh,
        scratch_types=[
            pltpu.VMEM((_K, _LANES), jnp.int32),
            pltpu.VMEM((_K, _LANES), jnp.int32),
            pltpu.VMEM((_LANES,), jnp.float32),
            pltpu.VMEM_SHARED((np_,), jnp.float32),
            pltpu.SemaphoreType.DMA,
            pltpu.SemaphoreType.DMA,
        ],
        compiler_params=pltpu.CompilerParams(use_tc_tiling_on_sc=False),
    )
    def deg_kernel(dst_hbm, zeros_hbm, out_hbm, idx_a, idx_b, ones_v,
                   deg_sh, sem_a, sem_b):
        c = lax.axis_index("c")
        s = lax.axis_index("s")
        pltpu.sync_copy(zeros_hbm.at[pl.ds(s * stretch, stretch)],
                        deg_sh.at[pl.ds(s * stretch, stretch)])
        for j in range(_LANES // 16):
            ones_v[pl.ds(j * 16, 16)] = jnp.ones((16,), jnp.float32)
        plsc.subcore_barrier()
        w = c * 16 + s
        wc0 = w * chunks

        def phase(g, cur_idx, nxt_idx, sem):
            ds = []
            for b in range(_K):
                ds.append(pltpu.async_copy(
                    ones_v, deg_sh.at[cur_idx.at[b]], sem, add=True))
            @pl.when(g + 1 < ngroups)
            def _():
                pltpu.sync_copy(dst_hbm.at[pl.ds(wc0 + (g + 1) * _K, _K)],
                                nxt_idx)
            for d in ds:
                d.wait()

        pltpu.sync_copy(dst_hbm.at[pl.ds(wc0, _K)], idx_a)

        def body(i, carry):
            phase(2 * i, idx_a, idx_b, sem_a)
            phase(2 * i + 1, idx_b, idx_a, sem_b)
            return carry

        lax.fori_loop(0, ngroups // 2, body, 0)
        plsc.subcore_barrier()
        pltpu.sync_copy(deg_sh.at[pl.ds(s * stretch, stretch)],
                        out_hbm.at[pl.ds(c * np_ + s * stretch, stretch)])

    return deg_kernel


def _sc_msg_kernel(np_, epw, chunks, stretch):
    mesh = plsc.VectorSubcoreMesh(core_axis_name="c", subcore_axis_name="s")
    ngroups = chunks // _K

    @functools.partial(
        pl.kernel,
        out_type=jax.ShapeDtypeStruct((4 * np_,), jnp.float32),
        mesh=mesh,
        scratch_types=[
            pltpu.VMEM((_K, _LANES), jnp.int32),
            pltpu.VMEM((_K, _LANES), jnp.int32),
            pltpu.VMEM((_K, _LANES), jnp.int32),
            pltpu.VMEM((_K, _LANES), jnp.int32),
            pltpu.VMEM((_K, _LANES), jnp.float32),
            pltpu.VMEM((_K, _LANES), jnp.float32),
            pltpu.VMEM((_K, _LANES), jnp.float32),
            pltpu.VMEM((_K, _LANES), jnp.float32),
            pltpu.VMEM_SHARED((np_,), jnp.float32),
            pltpu.VMEM_SHARED((np_,), jnp.float32),
            pltpu.SemaphoreType.DMA,
            pltpu.SemaphoreType.DMA,
            pltpu.SemaphoreType.DMA,
            pltpu.SemaphoreType.DMA,
        ],
        compiler_params=pltpu.CompilerParams(use_tc_tiling_on_sc=False),
    )
    def msg_kernel(src_hbm, dst_hbm, y0_hbm, y1_hbm, zeros_hbm, out_hbm,
                   sidx_a, didx_a, sidx_b, didx_b, g0a, g1a, g0b, g1b,
                   s0_sh, s1_sh, semg_a, semg_b, sems_a, sems_b):
        c = lax.axis_index("c")
        s = lax.axis_index("s")
        pltpu.sync_copy(zeros_hbm.at[pl.ds(s * stretch, stretch)],
                        s0_sh.at[pl.ds(s * stretch, stretch)])
        pltpu.sync_copy(zeros_hbm.at[pl.ds(s * stretch, stretch)],
                        s1_sh.at[pl.ds(s * stretch, stretch)])
        plsc.subcore_barrier()
        w = c * 16 + s
        wc0 = w * chunks

        def fire_gathers(sidx, g0, g1, semg):
            for b in range(_K):
                pltpu.async_copy(y0_hbm.at[sidx.at[b]], g0.at[b], semg)
                pltpu.async_copy(y1_hbm.at[sidx.at[b]], g1.at[b], semg)

        def phase(g, sidx, didx, g0, g1, semg, nsidx, ndidx, ng0, ng1, nsemg,
                  sems):
            @pl.when(g + 1 < ngroups)
            def _():
                pltpu.sync_copy(src_hbm.at[pl.ds(wc0 + (g + 1) * _K, _K)],
                                nsidx)
                pltpu.sync_copy(dst_hbm.at[pl.ds(wc0 + (g + 1) * _K, _K)],
                                ndidx)
                fire_gathers(nsidx, ng0, ng1, nsemg)
            for b in range(_K):
                pltpu.make_async_copy(y0_hbm.at[sidx.at[b]], g0.at[b],
                                      semg).wait()
                pltpu.make_async_copy(y1_hbm.at[sidx.at[b]], g1.at[b],
                                      semg).wait()
            ds = []
            for b in range(_K):
                ds.append(pltpu.async_copy(g0.at[b], s0_sh.at[didx.at[b]],
                                           sems, add=True))
                ds.append(pltpu.async_copy(g1.at[b], s1_sh.at[didx.at[b]],
                                           sems, add=True))
            for d in ds:
                d.wait()

        pltpu.sync_copy(src_hbm.at[pl.ds(wc0, _K)], sidx_a)
        pltpu.sync_copy(dst_hbm.at[pl.ds(wc0, _K)], didx_a)
        fire_gathers(sidx_a, g0a, g1a, semg_a)

        def body(i, carry):
            phase(2 * i, sidx_a, didx_a, g0a, g1a, semg_a,
                  sidx_b, didx_b, g0b, g1b, semg_b, sems_a)
            phase(2 * i + 1, sidx_b, didx_b, g0b, g1b, semg_b,
                  sidx_a, didx_a, g0a, g1a, semg_a, sems_b)
            return carry

        lax.fori_loop(0, ngroups // 2, body, 0)
        plsc.subcore_barrier()
        base = c * 2 * np_
        pltpu.sync_copy(s0_sh.at[pl.ds(s * stretch, stretch)],
                        out_hbm.at[pl.ds(base + s * stretch, stretch)])
        pltpu.sync_copy(s1_sh.at[pl.ds(s * stretch, stretch)],
                        out_hbm.at[pl.ds(base + np_ + s * stretch, stretch)])

    return msg_kernel


def kernel(x, edge_index, W, b, gamma, beta, fcW, fcb):
    n, in_dim = x.shape
    out_dim = W.shape[1]
    e = edge_index.shape[1]

    np_ = _pad_up(n + 1, 2048)
    stretch = np_ // 16
    e_pad = _pad_up(e, _NW * _LANES * _K * 2)
    epw = e_pad // _NW
    chunks = epw // _LANES

    ei32 = edge_index.astype(jnp.int32)
    pad = e_pad - e
    src = jnp.concatenate([ei32[0], jnp.zeros((pad,), jnp.int32)])
    dst = jnp.concatenate([ei32[1], jnp.full((pad,), n, jnp.int32)])
    src = src.reshape(e_pad // _LANES, _LANES)
    dst = dst.reshape(e_pad // _LANES, _LANES)

    zeros1 = jnp.zeros((np_,), jnp.float32)

    deg_flat = _sc_deg_kernel(np_, epw, chunks, stretch)(dst, zeros1)
    deg_parts = deg_flat.reshape(2, np_)

    xT = jnp.pad(x.T, ((0, 0), (0, np_ - n)))

    def prep_body(degp_ref, xT_ref, yT_ref, dinv_ref):
        dsum = degp_ref[0:1, :] + degp_ref[1:2, :] + 1.0
        dinv = lax.rsqrt(dsum)
        dinv_ref[...] = dinv
        yT_ref[...] = xT_ref[...] * dinv

    yT, dinv_row = pl.pallas_call(
        prep_body,
        out_shape=[jax.ShapeDtypeStruct((in_dim, np_), jnp.float32),
                   jax.ShapeDtypeStruct((1, np_), jnp.float32)],
    )(deg_parts, xT)
    y0 = yT[0]
    y1 = yT[1]

    s_flat = _sc_msg_kernel(np_, epw, chunks, stretch)(src, dst, y0, y1, zeros1)
    sf = s_flat.reshape(4, np_)

    blk = 2048
    nblk = np_ // blk
    WT_col = W.T
    b_col = b.reshape(out_dim, 1)
    g_col = gamma.reshape(out_dim, 1)
    be_col = beta.reshape(out_dim, 1)
    fcb_row = fcb.reshape(1, out_dim)

    def _hT_block(sfr, yTr, dr, WTr, bcr):
        a0 = (sfr[0:1, :] + sfr[2:3, :] + yTr[0:1, :]) * dr[...]
        a1 = (sfr[1:2, :] + sfr[3:4, :] + yTr[1:2, :]) * dr[...]
        zT = a0 * WTr[:, 0:1] + a1 * WTr[:, 1:2] + bcr[...]
        return jnp.maximum(zT, 0.0)

    def stats_body(sfr, yTr, dr, WTr, bcr, mean_o, var_o, acc_s, acc_q):
        i = pl.program_id(0)
        hT = _hT_block(sfr, yTr, dr, WTr, bcr)
        lane = lax.broadcasted_iota(jnp.int32, (1, blk), 1) + i * blk
        hT = jnp.where(lane < n, hT, 0.0)
        ps = jnp.sum(hT, axis=1, keepdims=True)
        pq = jnp.sum(hT * hT, axis=1, keepdims=True)

        @pl.when(i == 0)
        def _():
            acc_s[...] = ps
            acc_q[...] = pq

        @pl.when(i > 0)
        def _():
            acc_s[...] += ps
            acc_q[...] += pq

        @pl.when(i == nblk - 1)
        def _():
            m = acc_s[...] * (1.0 / n)
            mean_o[...] = m
            var_o[...] = acc_q[...] * (1.0 / n) - m * m

    sf_spec = pl.BlockSpec((4, blk), lambda i: (0, i))
    yT_spec = pl.BlockSpec((in_dim, blk), lambda i: (0, i))
    dv_spec = pl.BlockSpec((1, blk), lambda i: (0, i))
    full = lambda shape: pl.BlockSpec(shape, lambda i: (0, 0))
    mean, var = pl.pallas_call(
        stats_body,
        grid=(nblk,),
        in_specs=[sf_spec, yT_spec, dv_spec,
                  full((out_dim, in_dim)), full((out_dim, 1))],
        out_specs=[full((out_dim, 1)), full((out_dim, 1))],
        out_shape=[jax.ShapeDtypeStruct((out_dim, 1), jnp.float32),
                   jax.ShapeDtypeStruct((out_dim, 1), jnp.float32)],
        scratch_shapes=[pltpu.VMEM((out_dim, 1), jnp.float32),
                        pltpu.VMEM((out_dim, 1), jnp.float32)],
        compiler_params=pltpu.CompilerParams(
            dimension_semantics=("arbitrary",)),
    )(sf, yT, dinv_row, WT_col, b_col)

    def out_body(sfr, yTr, dr, WTr, bcr, mr, vr, gr, ber, fcWr, fcbr,
                 o1, o2):
        hT = _hT_block(sfr, yTr, dr, WTr, bcr)
        scale = gr[...] * lax.rsqrt(vr[...] + 1e-5)
        hnT = (hT - mr[...]) * scale + ber[...]
        hf = lax.dot_general(hnT, fcWr[...], (((0,), (0,)), ((), ())),
                             preferred_element_type=jnp.float32)
        hf = hf + fcbr[...]
        o2[...] = hf
        m = jnp.max(hf, axis=1, keepdims=True)
        shifted = hf - m
        lse = jnp.log(jnp.sum(jnp.exp(shifted), axis=1, keepdims=True))
        o1[...] = shifted - lse

    out_spec = pl.BlockSpec((blk, out_dim), lambda i: (i, 0))
    out1, out2 = pl.pallas_call(
        out_body,
        grid=(nblk,),
        in_specs=[sf_spec, yT_spec, dv_spec,
                  full((out_dim, in_dim)), full((out_dim, 1)),
                  full((out_dim, 1)), full((out_dim, 1)),
                  full((out_dim, 1)), full((out_dim, 1)),
                  full((out_dim, out_dim)), full((1, out_dim))],
        out_specs=[out_spec, out_spec],
        out_shape=[jax.ShapeDtypeStruct((n, out_dim), jnp.float32),
                   jax.ShapeDtypeStruct((n, out_dim), jnp.float32)],
        compiler_params=pltpu.CompilerParams(
            dimension_semantics=("arbitrary",)),
    )(sf, yT, dinv_row, WT_col, b_col, mean, var, g_col, be_col,
      fcW, fcb_row)
    return out1, out2


# speedup vs baseline: 77.5415x; 1.3662x over previous
import functools

import jax
import jax.numpy as jnp
from jax import lax
from jax.experimental import pallas as pl
from jax.experimental.pallas import tpu as pltpu
from jax.experimental.pallas import tpu_sc as plsc

_LANES = 128
_NW = 32


def _pad_up(n, m):
    return (n + m - 1) // m * m


_K = 4


def _sc_deg_kernel(np_, epw, chunks, stretch):
    mesh = plsc.VectorSubcoreMesh(core_axis_name="c", subcore_axis_name="s")
    ngroups = chunks // _K

    @functools.partial(
        pl.kernel,
        out_type=jax.ShapeDtypeStruct((2 * np_,), jnp.float32),
        mesh=mesh,
        scratch_types=[
            pltpu.VMEM((_K, _LANES), jnp.int32),
            pltpu.VMEM((_K, _LANES), jnp.int32),
            pltpu.VMEM((_LANES,), jnp.float32),
            pltpu.VMEM_SHARED((np_,), jnp.float32),
            pltpu.SemaphoreType.DMA,
            pltpu.SemaphoreType.DMA,
        ],
        compiler_params=pltpu.CompilerParams(use_tc_tiling_on_sc=False),
    )
    def deg_kernel(dst_hbm, zeros_hbm, out_hbm, idx_a, idx_b, ones_v,
                   deg_sh, sem_a, sem_b):
        c = lax.axis_index("c")
        s = lax.axis_index("s")
        pltpu.sync_copy(zeros_hbm.at[pl.ds(s * stretch, stretch)],
                        deg_sh.at[pl.ds(s * stretch, stretch)])
        for j in range(_LANES // 16):
            ones_v[pl.ds(j * 16, 16)] = jnp.ones((16,), jnp.float32)
        plsc.subcore_barrier()
        w = c * 16 + s
        wc0 = w * chunks

        def phase(g, cur_idx, nxt_idx, sem):
            ds = []
            for b in range(_K):
                ds.append(pltpu.async_copy(
                    ones_v, deg_sh.at[cur_idx.at[b]], sem, add=True))
            @pl.when(g + 1 < ngroups)
            def _():
                pltpu.sync_copy(dst_hbm.at[pl.ds(wc0 + (g + 1) * _K, _K)],
                                nxt_idx)
            for d in ds:
                d.wait()

        pltpu.sync_copy(dst_hbm.at[pl.ds(wc0, _K)], idx_a)

        def body(i, carry):
            phase(2 * i, idx_a, idx_b, sem_a)
            phase(2 * i + 1, idx_b, idx_a, sem_b)
            return carry

        lax.fori_loop(0, ngroups // 2, body, 0)
        plsc.subcore_barrier()
        pltpu.sync_copy(deg_sh.at[pl.ds(s * stretch, stretch)],
                        out_hbm.at[pl.ds(c * np_ + s * stretch, stretch)])

    return deg_kernel


def _sc_msg_kernel(np_, epw, chunks, stretch):
    mesh = plsc.VectorSubcoreMesh(core_axis_name="c", subcore_axis_name="s")
    ngroups = chunks // _K

    @functools.partial(
        pl.kernel,
        out_type=jax.ShapeDtypeStruct((4 * np_,), jnp.float32),
        mesh=mesh,
        scratch_types=[
            pltpu.VMEM((_K, _LANES), jnp.int32),
            pltpu.VMEM((_K, _LANES), jnp.int32),
            pltpu.VMEM((_K, _LANES), jnp.int32),
            pltpu.VMEM((_K, _LANES), jnp.int32),
            pltpu.VMEM((_K, _LANES), jnp.float32),
            pltpu.VMEM((_K, _LANES), jnp.float32),
            pltpu.VMEM((_K, _LANES), jnp.float32),
            pltpu.VMEM((_K, _LANES), jnp.float32),
            pltpu.VMEM_SHARED((np_,), jnp.float32),
            pltpu.VMEM_SHARED((np_,), jnp.float32),
            pltpu.SemaphoreType.DMA,
            pltpu.SemaphoreType.DMA,
            pltpu.SemaphoreType.DMA,
            pltpu.SemaphoreType.DMA,
        ],
        compiler_params=pltpu.CompilerParams(use_tc_tiling_on_sc=False),
    )
    def msg_kernel(src_hbm, dst_hbm, y0_hbm, y1_hbm, zeros_hbm, out_hbm,
                   sidx_a, didx_a, sidx_b, didx_b, g0a, g1a, g0b, g1b,
                   s0_sh, s1_sh, semg_a, semg_b, sems_a, sems_b):
        c = lax.axis_index("c")
        s = lax.axis_index("s")
        pltpu.sync_copy(zeros_hbm.at[pl.ds(s * stretch, stretch)],
                        s0_sh.at[pl.ds(s * stretch, stretch)])
        pltpu.sync_copy(zeros_hbm.at[pl.ds(s * stretch, stretch)],
                        s1_sh.at[pl.ds(s * stretch, stretch)])
        plsc.subcore_barrier()
        w = c * 16 + s
        wc0 = w * chunks

        def fire_gathers(sidx, g0, g1, semg):
            for b in range(_K):
                pltpu.async_copy(y0_hbm.at[sidx.at[b]], g0.at[b], semg)
                pltpu.async_copy(y1_hbm.at[sidx.at[b]], g1.at[b], semg)

        def phase(g, sidx, didx, g0, g1, semg, nsidx, ndidx, ng0, ng1, nsemg,
                  sems):
            @pl.when(g + 1 < ngroups)
            def _():
                pltpu.sync_copy(src_hbm.at[pl.ds(wc0 + (g + 1) * _K, _K)],
                                nsidx)
                pltpu.sync_copy(dst_hbm.at[pl.ds(wc0 + (g + 1) * _K, _K)],
                                ndidx)
                fire_gathers(nsidx, ng0, ng1, nsemg)
            for b in range(_K):
                pltpu.make_async_copy(y0_hbm.at[sidx.at[b]], g0.at[b],
                                      semg).wait()
                pltpu.make_async_copy(y1_hbm.at[sidx.at[b]], g1.at[b],
                                      semg).wait()
            ds = []
            for b in range(_K):
                ds.append(pltpu.async_copy(g0.at[b], s0_sh.at[didx.at[b]],
                                           sems, add=True))
                ds.append(pltpu.async_copy(g1.at[b], s1_sh.at[didx.at[b]],
                                           sems, add=True))
            for d in ds:
                d.wait()

        pltpu.sync_copy(src_hbm.at[pl.ds(wc0, _K)], sidx_a)
        pltpu.sync_copy(dst_hbm.at[pl.ds(wc0, _K)], didx_a)
        fire_gathers(sidx_a, g0a, g1a, semg_a)

        def body(i, carry):
            phase(2 * i, sidx_a, didx_a, g0a, g1a, semg_a,
                  sidx_b, didx_b, g0b, g1b, semg_b, sems_a)
            phase(2 * i + 1, sidx_b, didx_b, g0b, g1b, semg_b,
                  sidx_a, didx_a, g0a, g1a, semg_a, sems_b)
            return carry

        lax.fori_loop(0, ngroups // 2, body, 0)
        plsc.subcore_barrier()
        base = c * 2 * np_
        pltpu.sync_copy(s0_sh.at[pl.ds(s * stretch, stretch)],
                        out_hbm.at[pl.ds(base + s * stretch, stretch)])
        pltpu.sync_copy(s1_sh.at[pl.ds(s * stretch, stretch)],
                        out_hbm.at[pl.ds(base + np_ + s * stretch, stretch)])

    return msg_kernel


def kernel(x, edge_index, W, b, gamma, beta, fcW, fcb):
    n, in_dim = x.shape
    out_dim = W.shape[1]
    e = edge_index.shape[1]

    np_ = _pad_up(n + 1, 2048)
    stretch = np_ // 16
    e_pad = _pad_up(e, _NW * _LANES * _K * 2)
    epw = e_pad // _NW
    chunks = epw // _LANES

    pad = e_pad - e
    if edge_index.dtype == jnp.int64:
        pairs = lax.bitcast_convert_type(edge_index, jnp.int32)
        s32, d32 = pairs[0, :, 0], pairs[1, :, 0]
    else:
        s32, d32 = edge_index[0], edge_index[1]
    src = jnp.concatenate([s32, jnp.zeros((pad,), jnp.int32)])
    dst = jnp.concatenate([d32, jnp.full((pad,), n, jnp.int32)])
    src = src.reshape(e_pad // _LANES, _LANES)
    dst = dst.reshape(e_pad // _LANES, _LANES)

    zeros1 = jnp.zeros((np_,), jnp.float32)

    deg_flat = _sc_deg_kernel(np_, epw, chunks, stretch)(dst, zeros1)
    deg_parts = deg_flat.reshape(2, np_)

    xT = jnp.pad(x.T, ((0, 0), (0, np_ - n)))

    def prep_body(degp_ref, xT_ref, yT_ref, dinv_ref):
        dsum = degp_ref[0:1, :] + degp_ref[1:2, :] + 1.0
        dinv = lax.rsqrt(dsum)
        dinv_ref[...] = dinv
        yT_ref[...] = xT_ref[...] * dinv

    yT, dinv_row = pl.pallas_call(
        prep_body,
        out_shape=[jax.ShapeDtypeStruct((in_dim, np_), jnp.float32),
                   jax.ShapeDtypeStruct((1, np_), jnp.float32)],
    )(deg_parts, xT)
    y0 = yT[0]
    y1 = yT[1]

    s_flat = _sc_msg_kernel(np_, epw, chunks, stretch)(src, dst, y0, y1, zeros1)
    sf = s_flat.reshape(4, np_)

    blk = 2048
    nblk = np_ // blk
    WT_col = W.T
    b_col = b.reshape(out_dim, 1)
    g_col = gamma.reshape(out_dim, 1)
    be_col = beta.reshape(out_dim, 1)
    fcb_row = fcb.reshape(1, out_dim)

    def _hT_block(sfr, yTr, dr, WTr, bcr):
        a0 = (sfr[0:1, :] + sfr[2:3, :] + yTr[0:1, :]) * dr[...]
        a1 = (sfr[1:2, :] + sfr[3:4, :] + yTr[1:2, :]) * dr[...]
        zT = a0 * WTr[:, 0:1] + a1 * WTr[:, 1:2] + bcr[...]
        return jnp.maximum(zT, 0.0)

    def stats_body(sfr, yTr, dr, WTr, bcr, mean_o, var_o, acc_s, acc_q):
        i = pl.program_id(0)
        hT = _hT_block(sfr, yTr, dr, WTr, bcr)
        lane = lax.broadcasted_iota(jnp.int32, (1, blk), 1) + i * blk
        hT = jnp.where(lane < n, hT, 0.0)
        ps = jnp.sum(hT, axis=1, keepdims=True)
        pq = jnp.sum(hT * hT, axis=1, keepdims=True)

        @pl.when(i == 0)
        def _():
            acc_s[...] = ps
            acc_q[...] = pq

        @pl.when(i > 0)
        def _():
            acc_s[...] += ps
            acc_q[...] += pq

        @pl.when(i == nblk - 1)
        def _():
            m = acc_s[...] * (1.0 / n)
            mean_o[...] = m
            var_o[...] = acc_q[...] * (1.0 / n) - m * m

    sf_spec = pl.BlockSpec((4, blk), lambda i: (0, i))
    yT_spec = pl.BlockSpec((in_dim, blk), lambda i: (0, i))
    dv_spec = pl.BlockSpec((1, blk), lambda i: (0, i))
    full = lambda shape: pl.BlockSpec(shape, lambda i: (0, 0))
    mean, var = pl.pallas_call(
        stats_body,
        grid=(nblk,),
        in_specs=[sf_spec, yT_spec, dv_spec,
                  full((out_dim, in_dim)), full((out_dim, 1))],
        out_specs=[full((out_dim, 1)), full((out_dim, 1))],
        out_shape=[jax.ShapeDtypeStruct((out_dim, 1), jnp.float32),
                   jax.ShapeDtypeStruct((out_dim, 1), jnp.float32)],
        scratch_shapes=[pltpu.VMEM((out_dim, 1), jnp.float32),
                        pltpu.VMEM((out_dim, 1), jnp.float32)],
        compiler_params=pltpu.CompilerParams(
            dimension_semantics=("arbitrary",)),
    )(sf, yT, dinv_row, WT_col, b_col)

    fcb_col = fcb.reshape(out_dim, 1)

    def out_body(sfr, yTr, dr, WTr, bcr, mr, vr, gr, ber, fcWr, fcbr,
                 o1, o2):
        hT = _hT_block(sfr, yTr, dr, WTr, bcr)
        scale = gr[...] * lax.rsqrt(vr[...] + 1e-5)
        hnT = (hT - mr[...]) * scale + ber[...]
        hfT = lax.dot_general(fcWr[...], hnT, (((0,), (0,)), ((), ())),
                              preferred_element_type=jnp.float32)
        hfT = hfT + fcbr[...]
        o2[...] = hfT
        m = jnp.max(hfT, axis=0, keepdims=True)
        shifted = hfT - m
        lse = jnp.log(jnp.sum(jnp.exp(shifted), axis=0, keepdims=True))
        o1[...] = shifted - lse

    out_spec = pl.BlockSpec((out_dim, blk), lambda i: (0, i))
    out1T, out2T = pl.pallas_call(
        out_body,
        grid=(nblk,),
        in_specs=[sf_spec, yT_spec, dv_spec,
                  full((out_dim, in_dim)), full((out_dim, 1)),
                  full((out_dim, 1)), full((out_dim, 1)),
                  full((out_dim, 1)), full((out_dim, 1)),
                  full((out_dim, out_dim)), full((out_dim, 1))],
        out_specs=[out_spec, out_spec],
        out_shape=[jax.ShapeDtypeStruct((out_dim, n), jnp.float32),
                   jax.ShapeDtypeStruct((out_dim, n), jnp.float32)],
        compiler_params=pltpu.CompilerParams(
            dimension_semantics=("arbitrary",)),
    )(sf, yT, dinv_row, WT_col, b_col, mean, var, g_col, be_col,
      fcW, fcb_col)
    return out1T.T, out2T.T
